# baseline (speedup 1.0000x reference)
import functools
import math

import jax
import jax.numpy as jnp
from jax import lax
from jax.experimental import pallas as pl
from jax.experimental.pallas import tpu as pltpu
from jax.experimental.pallas import tpu_sc as plsc

D_MODEL = 64
D_PAD = 128
SCALE = math.sqrt(D_MODEL)
LANES = 16

NUM_CORES = 2
NUM_SUBCORES = 16
NUM_WORKERS = NUM_CORES * NUM_SUBCORES

CHUNK = 128
NBUF = 5


PREP_V = 2048


def _prep_body(wt_ref, out_ref):
    eye = jnp.eye(D_MODEL, dtype=jnp.float32) * SCALE
    out_ref[:, :D_MODEL] = jax.lax.dot_general(
        wt_ref[...],
        eye,
        dimension_numbers=(((0,), (0,)), ((), ())),
        preferred_element_type=jnp.float32,
    )


@jax.jit
def _prep_tc(wT):
    v = wT.shape[1]
    return pl.pallas_call(
        _prep_body,
        grid=((v + PREP_V - 1) // PREP_V,),
        in_specs=[pl.BlockSpec((D_MODEL, PREP_V), lambda i: (0, i))],
        out_specs=pl.BlockSpec((PREP_V, D_PAD), lambda i: (i, 0)),
        out_shape=jax.ShapeDtypeStruct((v, D_PAD), jnp.float32),
    )(wT)


@jax.jit
def _embed_sc(x3d, wpad):
    nw, n_chunks, _ = x3d.shape
    n_total = nw * n_chunks * CHUNK
    assert n_chunks % NBUF == 0

    mesh = plsc.VectorSubcoreMesh(core_axis_name="c", subcore_axis_name="s")

    @functools.partial(
        pl.kernel,
        out_type=jax.ShapeDtypeStruct((n_total, D_PAD), jnp.float32),
        mesh=mesh,
        scratch_types=[
            pltpu.VMEM((n_chunks, CHUNK), jnp.int32),
            [pltpu.VMEM((CHUNK, D_PAD), jnp.float32) for _ in range(NBUF)],
            pltpu.SemaphoreType.DMA,
            pltpu.SemaphoreType.DMA,
        ],
        compiler_params=pltpu.CompilerParams(use_tc_tiling_on_sc=False),
    )
    def body(w_hbm, idx_hbm, out_hbm, idx_v, rows, gsem, osem):
        wid = lax.axis_index("s") * NUM_CORES + lax.axis_index("c")
        base = wid * n_chunks * CHUNK
        pltpu.sync_copy(idx_hbm.at[wid], idx_v)

        @pl.loop(0, n_chunks, step=NBUF)
        def group_loop(c0):
            gets = [
                pltpu.async_copy(
                    w_hbm.at[idx_v.at[c0 + k]], rows[k], gsem
                )
                for k in range(NBUF)
            ]
            puts = []
            for k in range(NBUF):
                gets[k].wait()
                puts.append(
                    pltpu.async_copy(
                        rows[k],
                        out_hbm.at[pl.ds(base + (c0 + k) * CHUNK, CHUNK)],
                        osem,
                    )
                )
            for p in puts:
                p.wait()

    return body(wpad, x3d)


def kernel(x, weight):
    b, t = x.shape
    n = b * t
    n_per_w = n // NUM_WORKERS
    n_chunks = n_per_w // CHUNK
    x3d = x.reshape(NUM_WORKERS, n_chunks, CHUNK).astype(jnp.int32)
    wpad = _prep_tc(weight.T)
    outp = _embed_sc(x3d, wpad)
    return outp.reshape(b, t, D_PAD)[:, :, :D_MODEL]


# speedup vs baseline: 1.5445x; 1.5445x over previous
import functools
import math

import jax
import jax.numpy as jnp
from jax import lax
from jax.experimental import pallas as pl
from jax.experimental.pallas import tpu as pltpu
from jax.experimental.pallas import tpu_sc as plsc

D_MODEL = 64
D_PAD = 128
SCALE = math.sqrt(D_MODEL)
LANES = 16

NUM_CORES = 2
NUM_SUBCORES = 16
NUM_WORKERS = NUM_CORES * NUM_SUBCORES

CHUNK = 128
NBUF = 5


PREP_V = 8192


def _prep_body(wt_ref, out_ref):
    out_ref[:, :D_MODEL] = wt_ref[...].T * SCALE


@jax.jit
def _prep_tc(wT):
    v = wT.shape[1]
    return pl.pallas_call(
        _prep_body,
        grid=((v + PREP_V - 1) // PREP_V,),
        in_specs=[pl.BlockSpec((D_MODEL, PREP_V), lambda i: (0, i))],
        out_specs=pl.BlockSpec((PREP_V, D_PAD), lambda i: (i, 0)),
        out_shape=jax.ShapeDtypeStruct((v, D_PAD), jnp.float32),
    )(wT)


@jax.jit
def _embed_sc(x3d, wpad):
    nw, n_chunks, _ = x3d.shape
    n_total = nw * n_chunks * CHUNK
    assert n_chunks % NBUF == 0

    mesh = plsc.VectorSubcoreMesh(core_axis_name="c", subcore_axis_name="s")

    @functools.partial(
        pl.kernel,
        out_type=jax.ShapeDtypeStruct((n_total, D_PAD), jnp.float32),
        mesh=mesh,
        scratch_types=[
            pltpu.VMEM((n_chunks, CHUNK), jnp.int32),
            [pltpu.VMEM((CHUNK, D_PAD), jnp.float32) for _ in range(NBUF)],
            pltpu.SemaphoreType.DMA,
            pltpu.SemaphoreType.DMA,
        ],
        compiler_params=pltpu.CompilerParams(use_tc_tiling_on_sc=False),
    )
    def body(w_hbm, idx_hbm, out_hbm, idx_v, rows, gsem, osem):
        wid = lax.axis_index("s") * NUM_CORES + lax.axis_index("c")
        base = wid * n_chunks * CHUNK
        pltpu.sync_copy(idx_hbm.at[wid], idx_v)

        @pl.loop(0, n_chunks, step=NBUF)
        def group_loop(c0):
            gets = [
                pltpu.async_copy(
                    w_hbm.at[idx_v.at[c0 + k]], rows[k], gsem
                )
                for k in range(NBUF)
            ]
            puts = []
            for k in range(NBUF):
                gets[k].wait()
                puts.append(
                    pltpu.async_copy(
                        rows[k],
                        out_hbm.at[pl.ds(base + (c0 + k) * CHUNK, CHUNK)],
                        osem,
                    )
                )
            for p in puts:
                p.wait()

    return body(wpad, x3d)


def kernel(x, weight):
    b, t = x.shape
    n = b * t
    n_per_w = n // NUM_WORKERS
    n_chunks = n_per_w // CHUNK
    x3d = x.reshape(NUM_WORKERS, n_chunks, CHUNK).astype(jnp.int32)
    wpad = _prep_tc(weight.T)
    outp = _embed_sc(x3d, wpad)
    return outp.reshape(b, t, D_PAD)[:, :, :D_MODEL]


# speedup vs baseline: 1.6178x; 1.0474x over previous
import functools
import math

import jax
import jax.numpy as jnp
from jax import lax
from jax.experimental import pallas as pl
from jax.experimental.pallas import tpu as pltpu
from jax.experimental.pallas import tpu_sc as plsc

D_MODEL = 64
D_PAD = 128
SCALE = math.sqrt(D_MODEL)
LANES = 16

NUM_CORES = 2
NUM_SUBCORES = 16
NUM_WORKERS = NUM_CORES * NUM_SUBCORES

CHUNK = 128
NBUF = 5


PREP_V = 16384


def _prep_body(wt_ref, out_ref):
    out_ref[:, :D_MODEL] = wt_ref[...].T * SCALE


@jax.jit
def _prep_tc(wT):
    v = wT.shape[1]
    return pl.pallas_call(
        _prep_body,
        grid=((v + PREP_V - 1) // PREP_V,),
        in_specs=[pl.BlockSpec((D_MODEL, PREP_V), lambda i: (0, i))],
        out_specs=pl.BlockSpec((PREP_V, D_PAD), lambda i: (i, 0)),
        out_shape=jax.ShapeDtypeStruct((v, D_PAD), jnp.float32),
    )(wT)


@jax.jit
def _embed_sc(x3d, wpad):
    nw, n_chunks, _ = x3d.shape
    n_total = nw * n_chunks * CHUNK
    assert n_chunks % NBUF == 0

    mesh = plsc.VectorSubcoreMesh(core_axis_name="c", subcore_axis_name="s")

    @functools.partial(
        pl.kernel,
        out_type=jax.ShapeDtypeStruct((n_total, D_PAD), jnp.float32),
        mesh=mesh,
        scratch_types=[
            pltpu.VMEM((n_chunks, CHUNK), jnp.int32),
            [pltpu.VMEM((CHUNK, D_PAD), jnp.float32) for _ in range(NBUF)],
            pltpu.SemaphoreType.DMA,
            pltpu.SemaphoreType.DMA,
        ],
        compiler_params=pltpu.CompilerParams(use_tc_tiling_on_sc=False),
    )
    def body(w_hbm, idx_hbm, out_hbm, idx_v, rows, gsem, osem):
        wid = lax.axis_index("s") * NUM_CORES + lax.axis_index("c")
        base = wid * n_chunks * CHUNK
        pltpu.sync_copy(idx_hbm.at[wid], idx_v)

        @pl.loop(0, n_chunks, step=NBUF)
        def group_loop(c0):
            gets = [
                pltpu.async_copy(
                    w_hbm.at[idx_v.at[c0 + k]], rows[k], gsem
                )
                for k in range(NBUF)
            ]
            puts = []
            for k in range(NBUF):
                gets[k].wait()
                puts.append(
                    pltpu.async_copy(
                        rows[k],
                        out_hbm.at[pl.ds(base + (c0 + k) * CHUNK, CHUNK)],
                        osem,
                    )
                )
            for p in puts:
                p.wait()

    return body(wpad, x3d)


def kernel(x, weight):
    b, t = x.shape
    n = b * t
    n_per_w = n // NUM_WORKERS
    n_chunks = n_per_w // CHUNK
    x3d = x.reshape(NUM_WORKERS, n_chunks, CHUNK).astype(jnp.int32)
    wpad = _prep_tc(weight.T)
    outp = _embed_sc(x3d, wpad)
    return outp.reshape(b, t, D_PAD)[:, :, :D_MODEL]


# speedup vs baseline: 1.6506x; 1.0203x over previous
import functools
import math

import jax
import jax.numpy as jnp
from jax import lax
from jax.experimental import pallas as pl
from jax.experimental.pallas import tpu as pltpu
from jax.experimental.pallas import tpu_sc as plsc

D_MODEL = 64
D_PAD = 128
SCALE = math.sqrt(D_MODEL)
LANES = 16

NUM_CORES = 2
NUM_SUBCORES = 16
NUM_WORKERS = NUM_CORES * NUM_SUBCORES

CHUNK = 128
NBUF = 5


PREP_V = 32768


def _prep_body(wt_ref, out_ref):
    out_ref[:, :D_MODEL] = wt_ref[...].T * SCALE


@jax.jit
def _prep_tc(wT):
    v = wT.shape[1]
    return pl.pallas_call(
        _prep_body,
        grid=((v + PREP_V - 1) // PREP_V,),
        in_specs=[pl.BlockSpec((D_MODEL, PREP_V), lambda i: (0, i))],
        out_specs=pl.BlockSpec((PREP_V, D_PAD), lambda i: (i, 0)),
        out_shape=jax.ShapeDtypeStruct((v, D_PAD), jnp.float32),
    )(wT)


@jax.jit
def _embed_sc(x3d, wpad):
    nw, n_chunks, _ = x3d.shape
    n_total = nw * n_chunks * CHUNK
    assert n_chunks % NBUF == 0

    mesh = plsc.VectorSubcoreMesh(core_axis_name="c", subcore_axis_name="s")

    @functools.partial(
        pl.kernel,
        out_type=jax.ShapeDtypeStruct((n_total, D_PAD), jnp.float32),
        mesh=mesh,
        scratch_types=[
            pltpu.VMEM((n_chunks, CHUNK), jnp.int32),
            [pltpu.VMEM((CHUNK, D_PAD), jnp.float32) for _ in range(NBUF)],
            pltpu.SemaphoreType.DMA,
            pltpu.SemaphoreType.DMA,
        ],
        compiler_params=pltpu.CompilerParams(use_tc_tiling_on_sc=False),
    )
    def body(w_hbm, idx_hbm, out_hbm, idx_v, rows, gsem, osem):
        wid = lax.axis_index("s") * NUM_CORES + lax.axis_index("c")
        base = wid * n_chunks * CHUNK
        pltpu.sync_copy(idx_hbm.at[wid], idx_v)

        @pl.loop(0, n_chunks, step=NBUF)
        def group_loop(c0):
            gets = [
                pltpu.async_copy(
                    w_hbm.at[idx_v.at[c0 + k]], rows[k], gsem
                )
                for k in range(NBUF)
            ]
            puts = []
            for k in range(NBUF):
                gets[k].wait()
                puts.append(
                    pltpu.async_copy(
                        rows[k],
                        out_hbm.at[pl.ds(base + (c0 + k) * CHUNK, CHUNK)],
                        osem,
                    )
                )
            for p in puts:
                p.wait()

    return body(wpad, x3d)


def kernel(x, weight):
    b, t = x.shape
    n = b * t
    n_per_w = n // NUM_WORKERS
    n_chunks = n_per_w // CHUNK
    x3d = x.reshape(NUM_WORKERS, n_chunks, CHUNK).astype(jnp.int32)
    wpad = _prep_tc(weight.T)
    outp = _embed_sc(x3d, wpad)
    return outp.reshape(b, t, D_PAD)[:, :, :D_MODEL]
